# baseline (speedup 1.0000x reference)
import functools

import jax
import jax.numpy as jnp
from jax import lax
from jax.experimental import pallas as pl
from jax.experimental.pallas import tpu as pltpu
from jax.experimental.pallas import tpu_sc as plsc

F32 = jnp.float32

N = 10000
E = 320000
D = 128

NC = 2
NS = 16
NW = NC * NS
K = 80
CW = E // (NW * K)
EPW = CW * K
UNROLL = 6
CWU = (CW // UNROLL) * UNROLL
NPAD = 10240
NPAD2 = 10112
TPW = NPAD // NS
TPW2 = NPAD2 // NS

_mesh = plsc.VectorSubcoreMesh(core_axis_name="c", subcore_axis_name="s")


def _zero_rows(buf):
    z = jnp.zeros((16,), F32)

    def body(i, _):
        for k in range(buf.shape[1] // 16):
            buf[i, pl.ds(k * 16, 16)] = z
        return 0

    lax.fori_loop(0, buf.shape[0], body, 0)


def _zero_1d(buf):
    z = jnp.zeros((16,), F32)

    def body(i, _):
        buf[pl.ds(i * 16, 16)] = z
        return 0

    lax.fori_loop(0, buf.shape[0] // 16, body, 0)


@functools.partial(
    pl.kernel,
    out_type=jax.ShapeDtypeStruct((NC, NPAD), F32),
    mesh=_mesh,
    scratch_types=[
        pltpu.VMEM((2, K), jnp.int32),
        pltpu.VMEM((K,), F32),
        pltpu.VMEM((TPW,), F32),
        pltpu.VMEM_SHARED((NPAD,), F32),
        pltpu.SemaphoreType.DMA,
        pltpu.SemaphoreType.DMA,
    ],
)
def _deg(col_hbm, out_hbm, cibuf, ones, zbuf, dacc, i0, i1):
    c = lax.axis_index("c")
    s = lax.axis_index("s")
    wid = s * NC + c
    isem = (i0, i1)

    def idx_cp(j, m):
        off = pl.multiple_of(wid * EPW + j * K, 8)
        return pltpu.make_async_copy(col_hbm.at[pl.ds(off, K)], cibuf.at[m],
                                     isem[m])

    for k in range(K // 16):
        ones[pl.ds(k * 16, 16)] = jnp.ones((16,), F32)
    _zero_1d(zbuf)
    pltpu.sync_copy(zbuf, dacc.at[pl.ds(s * TPW, TPW)])
    plsc.subcore_barrier()

    idx_cp(0, 0).start()

    def body(j, _):
        for m in range(2):
            jj = 2 * j + m

            @pl.when(jj + 1 < CW)
            def _():
                idx_cp(jj + 1, (m + 1) % 2).start()

            idx_cp(jj, m).wait()
            pltpu.sync_copy(ones, dacc.at[cibuf.at[m]], add=True)
        return 0

    lax.fori_loop(0, CW // 2, body, 0)
    if CW % 2 == 1:
        idx_cp(CW - 1, (CW - 1) % 2).wait()
        pltpu.sync_copy(ones, dacc.at[cibuf.at[(CW - 1) % 2]], add=True)
    plsc.subcore_barrier()
    pltpu.sync_copy(dacc.at[pl.ds(s * TPW, TPW)], out_hbm.at[c, pl.ds(s * TPW, TPW)])


def _agg_body(with_wsum, g_hbm, row_hbm, col_hbm, *args):
    if with_wsum:
        (dinv_hbm, out_hbm, wout_hbm, ribuf, cibuf, b0, b1, b2, acc, dsp,
         wacc, vals, zbuf, s0, s1, s2, i0, i1, i2, i3, i4, i5,
         c0, c1, c2) = args
    else:
        (out_hbm, ribuf, cibuf, b0, b1, b2, acc, s0, s1, s2,
         i0, i1, i2, i3, i4, i5, c0, c1, c2) = args
    bufs = (b0, b1, b2)
    gsem = (s0, s1, s2)
    isem = (i0, i1, i2, i3, i4, i5)
    ssem = (c0, c1, c2)
    c = lax.axis_index("c")
    s = lax.axis_index("s")
    wid = s * NC + c

    _zero_rows(b0)
    nfull = TPW2 // K
    for b in range(nfull):
        pltpu.sync_copy(b0, acc.at[pl.ds(s * TPW2 + b * K, K)])
    rem = TPW2 - nfull * K
    if rem:
        pltpu.sync_copy(b0.at[pl.ds(0, rem)],
                        acc.at[pl.ds(s * TPW2 + nfull * K, rem)])
    if with_wsum:
        _zero_1d(zbuf)
        pltpu.sync_copy(zbuf, wacc.at[pl.ds(s * TPW, TPW)])

        @pl.when(s == 0)
        def _():
            pltpu.sync_copy(dinv_hbm, dsp)

    plsc.subcore_barrier()

    def _ioff(j):
        return pl.multiple_of(wid * EPW + j * K, 8)

    def idx_start(j, m):
        off = _ioff(j)
        pltpu.make_async_copy(row_hbm.at[pl.ds(off, K)], ribuf.at[m], isem[m]).start()
        pltpu.make_async_copy(col_hbm.at[pl.ds(off, K)], cibuf.at[m], isem[m]).start()

    def idx_wait(j, m):
        off = _ioff(j)
        pltpu.make_async_copy(row_hbm.at[pl.ds(off, K)], ribuf.at[m], isem[m]).wait()
        pltpu.make_async_copy(col_hbm.at[pl.ds(off, K)], cibuf.at[m], isem[m]).wait()

    def g_cp(t, m):
        return pltpu.make_async_copy(g_hbm.at[ribuf.at[m]], bufs[t], gsem[t])

    def sc_start(t, m):
        pltpu.async_copy(bufs[t], acc.at[cibuf.at[m]], ssem[t], add=True)

    def sc_wait(t, m):
        pltpu.make_async_copy(bufs[t], acc.at[cibuf.at[m]], ssem[t]).wait()

    for j in range(4):
        idx_start(j, j)
    idx_wait(0, 0)
    g_cp(0, 0).start()
    idx_wait(1, 1)
    g_cp(1, 1).start()

    def step(j, u, cond):
        t = u % 3
        tn = (u + 2) % 3
        mn = (u + 2) % 6

        def ab():
            idx_wait(j + 2, mn)

            def sw():
                sc_wait(tn, (u + 5) % 6)

            cond(j >= 1, sw)
            g_cp(tn, mn).start()

        cond(j + 2 < CW, ab)
        g_cp(t, u).wait()
        sc_start(t, u)
        if with_wsum:
            pltpu.sync_copy(dsp.at[cibuf.at[u]], vals)
            pltpu.sync_copy(vals, wacc.at[ribuf.at[u]], add=True)
        cond(j + 4 < CW, lambda: idx_start(j + 4, (u + 4) % 6))

    def dyn_cond(pred, fn):
        pl.when(pred)(fn)

    def static_cond(pred, fn):
        if pred:
            fn()

    def body(k, _):
        j0 = UNROLL * k
        for u in range(UNROLL):
            step(j0 + u, u, dyn_cond)
        return 0

    lax.fori_loop(0, CWU // UNROLL, body, 0)
    for j in range(CWU, CW):
        step(j, j % UNROLL, static_cond)
    for j in range(CW - 3, CW):
        sc_wait(j % 3, j % 6)

    plsc.subcore_barrier()
    pltpu.sync_copy(acc.at[pl.ds(s * TPW2, TPW2)],
                    out_hbm.at[c, pl.ds(s * TPW2, TPW2)])
    if with_wsum:
        pltpu.sync_copy(wacc.at[pl.ds(s * TPW, TPW)],
                        wout_hbm.at[c, pl.ds(s * TPW, TPW)])


_agg = functools.partial(
    pl.kernel,
    out_type=jax.ShapeDtypeStruct((NC, NPAD2, D), F32),
    mesh=_mesh,
    scratch_types=[
        pltpu.VMEM((6, K), jnp.int32),
        pltpu.VMEM((6, K), jnp.int32),
        pltpu.VMEM((K, D), F32),
        pltpu.VMEM((K, D), F32),
        pltpu.VMEM((K, D), F32),
        pltpu.VMEM_SHARED((NPAD2, D), F32),
    ] + [pltpu.SemaphoreType.DMA] * 12,
)(functools.partial(_agg_body, False))

_aggw = functools.partial(
    pl.kernel,
    out_type=[jax.ShapeDtypeStruct((NC, NPAD2, D), F32),
              jax.ShapeDtypeStruct((NC, NPAD), F32)],
    mesh=_mesh,
    scratch_types=[
        pltpu.VMEM((6, K), jnp.int32),
        pltpu.VMEM((6, K), jnp.int32),
        pltpu.VMEM((K, D), F32),
        pltpu.VMEM((K, D), F32),
        pltpu.VMEM((K, D), F32),
        pltpu.VMEM_SHARED((NPAD2, D), F32),
        pltpu.VMEM_SHARED((N,), F32),
        pltpu.VMEM_SHARED((NPAD,), F32),
        pltpu.VMEM((K,), F32),
        pltpu.VMEM((TPW,), F32),
    ] + [pltpu.SemaphoreType.DMA] * 12,
)(functools.partial(_agg_body, True))


TB = 1000


def _dinv_from(dg_ref):
    deg = dg_ref[0] + dg_ref[1] + 1.0
    return lax.rsqrt(deg)


def _tc1_body(x_ref, w_ref, dg_ref, g_ref, dinv_ref):
    dinv = _dinv_from(dg_ref)
    h = jnp.dot(x_ref[...], w_ref[...], preferred_element_type=F32)
    g_ref[...] = h * dinv
    dinv_ref[...] = dinv


def _tc2_body(g_in_ref, p_ref, dg_ref, b_ref, w_ref, out_ref):
    dinv = _dinv_from(dg_ref)
    su = p_ref[0] + p_ref[1] + g_in_ref[...]
    h = jnp.maximum(dinv * su + b_ref[...], 0.0)
    out_ref[...] = dinv * jnp.dot(h, w_ref[...], preferred_element_type=F32)


def _tc3_body(g_in_ref, p_ref, dg_ref, wp_ref, b2_ref, w3_ref, b3_ref,
              out_ref, acc):
    i = pl.program_id(0)

    @pl.when(i == 0)
    def _():
        acc[...] = jnp.zeros((8, D), F32)

    dinv = _dinv_from(dg_ref)
    su = p_ref[0] + p_ref[1] + g_in_ref[...]
    h3 = jnp.maximum(dinv * su + b2_ref[...], 0.0)
    v = dinv * (wp_ref[0] + wp_ref[1] + dinv)
    acc[0:1, :] = acc[0:1, :] + jnp.sum(v * h3, axis=0, keepdims=True)

    @pl.when(i == pl.num_programs(0) - 1)
    def _():
        out_ref[...] = (
            jnp.dot(acc[0:1, :] / float(N), w3_ref[...],
                    preferred_element_type=F32) + b3_ref[...]
        )


def _row_spec(block):
    return pl.BlockSpec(block, lambda i: (i, 0))


def _const_spec(block):
    return pl.BlockSpec(block, lambda i: (0, 0))


_p_spec = pl.BlockSpec((2, TB, D), lambda i: (0, i, 0))
_v_spec = pl.BlockSpec((2, TB, 1), lambda i: (0, i, 0))

_tc1 = pl.pallas_call(
    _tc1_body,
    grid=(N // TB,),
    in_specs=[_row_spec((TB, D)), _const_spec((D, D)), _v_spec],
    out_specs=[_row_spec((TB, D)), _row_spec((TB, 1))],
    out_shape=[jax.ShapeDtypeStruct((N, D), F32),
               jax.ShapeDtypeStruct((N, 1), F32)],
)

_tc2 = pl.pallas_call(
    _tc2_body,
    grid=(N // TB,),
    in_specs=[_row_spec((TB, D)), _p_spec, _v_spec,
              _const_spec((1, D)), _const_spec((D, D))],
    out_specs=_row_spec((TB, D)),
    out_shape=jax.ShapeDtypeStruct((N, D), F32),
)

_tc3 = pl.pallas_call(
    _tc3_body,
    grid=(N // TB,),
    in_specs=[_row_spec((TB, D)), _p_spec, _v_spec, _v_spec,
              _const_spec((1, D)), _const_spec((D, D)), _const_spec((1, D))],
    out_specs=_const_spec((1, D)),
    out_shape=jax.ShapeDtypeStruct((1, D), F32),
    scratch_shapes=[pltpu.VMEM((8, D), F32)],
)


def kernel(x, edge_index, W1, b1, W2, b2, W3, b3):
    roww = edge_index[0]
    colw = edge_index[1]

    degp = _deg(colw).reshape(NC, NPAD, 1)

    g1, dinv = _tc1(x, W1, degp)

    p1 = _agg(g1, roww, colw)
    g2 = _tc2(g1, p1, degp, b1.reshape(1, D), W2)
    p2, wp = _aggw(g2, roww, colw, dinv.reshape(N))

    return _tc3(g2, p2, degp, wp.reshape(NC, NPAD, 1),
                b2.reshape(1, D), W3, b3.reshape(1, D))


# speedup vs baseline: 1.0378x; 1.0378x over previous
import functools

import jax
import jax.numpy as jnp
from jax import lax
from jax.experimental import pallas as pl
from jax.experimental.pallas import tpu as pltpu
from jax.experimental.pallas import tpu_sc as plsc

F32 = jnp.float32

N = 10000
E = 320000
D = 128

NC = 2
NS = 16
NW = NC * NS
EPW = E // NW
K = 112
CWF = EPW // K
KT = EPW - CWF * K
CW = CWF + 1
UNROLL = 6
CWU = 84
NPAD = 10240
NPAD2 = 10112
TPW = NPAD // NS
TPW2 = NPAD2 // NS

_mesh = plsc.VectorSubcoreMesh(core_axis_name="c", subcore_axis_name="s")


def _zero_rows(buf):
    z = jnp.zeros((16,), F32)

    def body(i, _):
        for k in range(buf.shape[1] // 16):
            buf[i, pl.ds(k * 16, 16)] = z
        return 0

    lax.fori_loop(0, buf.shape[0], body, 0)


def _zero_1d(buf):
    z = jnp.zeros((16,), F32)

    def body(i, _):
        buf[pl.ds(i * 16, 16)] = z
        return 0

    lax.fori_loop(0, buf.shape[0] // 16, body, 0)


@functools.partial(
    pl.kernel,
    out_type=jax.ShapeDtypeStruct((NC, NPAD), F32),
    mesh=_mesh,
    scratch_types=[
        pltpu.VMEM((2, K), jnp.int32),
        pltpu.VMEM((1, KT), jnp.int32),
        pltpu.VMEM((K,), F32),
        pltpu.VMEM((TPW,), F32),
        pltpu.VMEM_SHARED((NPAD,), F32),
        pltpu.SemaphoreType.DMA,
        pltpu.SemaphoreType.DMA,
    ],
)
def _deg(col_hbm, out_hbm, cibuf, ctail, ones, zbuf, dacc, i0, i1):
    c = lax.axis_index("c")
    s = lax.axis_index("s")
    wid = s * NC + c
    isem = (i0, i1)

    def idx_cp(j, m):
        off = pl.multiple_of(wid * EPW + j * K, 8)
        return pltpu.make_async_copy(col_hbm.at[pl.ds(off, K)], cibuf.at[m],
                                     isem[m])

    for k in range(K // 16):
        ones[pl.ds(k * 16, 16)] = jnp.ones((16,), F32)
    _zero_1d(zbuf)
    pltpu.sync_copy(zbuf, dacc.at[pl.ds(s * TPW, TPW)])
    plsc.subcore_barrier()

    idx_cp(0, 0).start()

    def body(j, _):
        for m in range(2):
            jj = 2 * j + m

            @pl.when(jj + 1 < CWF)
            def _():
                idx_cp(jj + 1, (m + 1) % 2).start()

            idx_cp(jj, m).wait()
            pltpu.sync_copy(ones, dacc.at[cibuf.at[m]], add=True)
        return 0

    lax.fori_loop(0, CWF // 2, body, 0)
    if CWF % 2 == 1:
        idx_cp(CWF - 1, (CWF - 1) % 2).wait()
        pltpu.sync_copy(ones, dacc.at[cibuf.at[(CWF - 1) % 2]], add=True)
    toff = pl.multiple_of(wid * EPW + CWF * K, 8)
    pltpu.sync_copy(col_hbm.at[pl.ds(toff, KT)], ctail.at[0])
    pltpu.sync_copy(ones.at[pl.ds(0, KT)], dacc.at[ctail.at[0]], add=True)
    plsc.subcore_barrier()
    pltpu.sync_copy(dacc.at[pl.ds(s * TPW, TPW)], out_hbm.at[c, pl.ds(s * TPW, TPW)])


def _agg_body(with_wsum, g_hbm, row_hbm, col_hbm, *args):
    if with_wsum:
        (dinv_hbm, out_hbm, wout_hbm, ribuf, cibuf, rtail, ctail, b0, b1, b2,
         acc, dsp, wacc, vals, zbuf, s0, s1, s2, i0, i1, i2, i3, i4, i5,
         tsem, c0, c1, c2) = args
    else:
        (out_hbm, ribuf, cibuf, rtail, ctail, b0, b1, b2, acc, s0, s1, s2,
         i0, i1, i2, i3, i4, i5, tsem, c0, c1, c2) = args
    bufs = (b0, b1, b2)
    gsem = (s0, s1, s2)
    isem = (i0, i1, i2, i3, i4, i5)
    ssem = (c0, c1, c2)
    c = lax.axis_index("c")
    s = lax.axis_index("s")
    wid = s * NC + c

    _zero_rows(b0)
    nfull = TPW2 // K
    for b in range(nfull):
        pltpu.sync_copy(b0, acc.at[pl.ds(s * TPW2 + b * K, K)])
    rem = TPW2 - nfull * K
    if rem:
        pltpu.sync_copy(b0.at[pl.ds(0, rem)],
                        acc.at[pl.ds(s * TPW2 + nfull * K, rem)])
    if with_wsum:
        _zero_1d(zbuf)
        pltpu.sync_copy(zbuf, wacc.at[pl.ds(s * TPW, TPW)])

        @pl.when(s == 0)
        def _():
            pltpu.sync_copy(dinv_hbm, dsp)

    plsc.subcore_barrier()

    def _ioff(j):
        return pl.multiple_of(wid * EPW + j * K, 8)

    def idx_start(j, m):
        off = _ioff(j)
        pltpu.make_async_copy(row_hbm.at[pl.ds(off, K)], ribuf.at[m], isem[m]).start()
        pltpu.make_async_copy(col_hbm.at[pl.ds(off, K)], cibuf.at[m], isem[m]).start()

    def idx_wait(j, m):
        off = _ioff(j)
        pltpu.make_async_copy(row_hbm.at[pl.ds(off, K)], ribuf.at[m], isem[m]).wait()
        pltpu.make_async_copy(col_hbm.at[pl.ds(off, K)], cibuf.at[m], isem[m]).wait()

    def g_cp(t, m):
        return pltpu.make_async_copy(g_hbm.at[ribuf.at[m]], bufs[t], gsem[t])

    def sc_start(t, m):
        pltpu.async_copy(bufs[t], acc.at[cibuf.at[m]], ssem[t], add=True)

    def sc_wait(t, m):
        pltpu.make_async_copy(bufs[t], acc.at[cibuf.at[m]], ssem[t]).wait()

    for j in range(4):
        idx_start(j, j)
    idx_wait(0, 0)
    g_cp(0, 0).start()
    idx_wait(1, 1)
    g_cp(1, 1).start()

    toff = pl.multiple_of(wid * EPW + CWF * K, 8)

    def tidx_start():
        pltpu.make_async_copy(row_hbm.at[pl.ds(toff, KT)], rtail.at[0], tsem).start()
        pltpu.make_async_copy(col_hbm.at[pl.ds(toff, KT)], ctail.at[0], tsem).start()

    def tidx_wait():
        pltpu.make_async_copy(row_hbm.at[pl.ds(toff, KT)], rtail.at[0], tsem).wait()
        pltpu.make_async_copy(col_hbm.at[pl.ds(toff, KT)], ctail.at[0], tsem).wait()

    def tg_cp(t):
        return pltpu.make_async_copy(g_hbm.at[rtail.at[0]],
                                     bufs[t].at[pl.ds(0, KT)], gsem[t])

    def tsc_start(t):
        pltpu.async_copy(bufs[t].at[pl.ds(0, KT)], acc.at[ctail.at[0]],
                         ssem[t], add=True)

    def tsc_wait(t):
        pltpu.make_async_copy(bufs[t].at[pl.ds(0, KT)], acc.at[ctail.at[0]],
                              ssem[t]).wait()

    def step(j, u, cond, gnext_tail=False, inext_tail=False):
        t = u % 3
        tn = (u + 2) % 3
        mn = (u + 2) % 6

        def ab():
            def sw():
                sc_wait(tn, (u + 5) % 6)

            if gnext_tail:
                tidx_wait()
                cond(j >= 1, sw)
                tg_cp(tn).start()
            else:
                idx_wait(j + 2, mn)
                cond(j >= 1, sw)
                g_cp(tn, mn).start()

        cond(j + 2 < CW, ab)
        g_cp(t, u).wait()
        sc_start(t, u)
        if with_wsum:
            pltpu.sync_copy(dsp.at[cibuf.at[u]], vals)
            pltpu.sync_copy(vals, wacc.at[ribuf.at[u]], add=True)
        if inext_tail:
            tidx_start()
        else:
            cond(j + 4 < CW - 1, lambda: idx_start(j + 4, (u + 4) % 6))

    def dyn_cond(pred, fn):
        pl.when(pred)(fn)

    def static_cond(pred, fn):
        if pred:
            fn()

    def body(k, _):
        j0 = UNROLL * k
        for u in range(UNROLL):
            step(j0 + u, u, dyn_cond)
        return 0

    lax.fori_loop(0, CWU // UNROLL, body, 0)
    for j in range(CWU, CWF):
        step(j, j % UNROLL, static_cond,
             gnext_tail=(j + 2 == CW - 1), inext_tail=(j + 4 == CW - 1))
    tt = (CW - 1) % 3
    tg_cp(tt).wait()
    tsc_start(tt)
    if with_wsum:
        pltpu.sync_copy(dsp.at[ctail.at[0]], vals.at[pl.ds(0, KT)])
        pltpu.sync_copy(vals.at[pl.ds(0, KT)], wacc.at[rtail.at[0]], add=True)
    for j in range(CW - 3, CW - 1):
        sc_wait(j % 3, j % 6)
    tsc_wait(tt)

    plsc.subcore_barrier()
    pltpu.sync_copy(acc.at[pl.ds(s * TPW2, TPW2)],
                    out_hbm.at[c, pl.ds(s * TPW2, TPW2)])
    if with_wsum:
        pltpu.sync_copy(wacc.at[pl.ds(s * TPW, TPW)],
                        wout_hbm.at[c, pl.ds(s * TPW, TPW)])


_agg = functools.partial(
    pl.kernel,
    out_type=jax.ShapeDtypeStruct((NC, NPAD2, D), F32),
    mesh=_mesh,
    scratch_types=[
        pltpu.VMEM((6, K), jnp.int32),
        pltpu.VMEM((6, K), jnp.int32),
        pltpu.VMEM((1, KT), jnp.int32),
        pltpu.VMEM((1, KT), jnp.int32),
        pltpu.VMEM((K, D), F32),
        pltpu.VMEM((K, D), F32),
        pltpu.VMEM((K, D), F32),
        pltpu.VMEM_SHARED((NPAD2, D), F32),
    ] + [pltpu.SemaphoreType.DMA] * 13,
)(functools.partial(_agg_body, False))

_aggw = functools.partial(
    pl.kernel,
    out_type=[jax.ShapeDtypeStruct((NC, NPAD2, D), F32),
              jax.ShapeDtypeStruct((NC, NPAD), F32)],
    mesh=_mesh,
    scratch_types=[
        pltpu.VMEM((6, K), jnp.int32),
        pltpu.VMEM((6, K), jnp.int32),
        pltpu.VMEM((1, KT), jnp.int32),
        pltpu.VMEM((1, KT), jnp.int32),
        pltpu.VMEM((K, D), F32),
        pltpu.VMEM((K, D), F32),
        pltpu.VMEM((K, D), F32),
        pltpu.VMEM_SHARED((NPAD2, D), F32),
        pltpu.VMEM_SHARED((N,), F32),
        pltpu.VMEM_SHARED((NPAD,), F32),
        pltpu.VMEM((K,), F32),
        pltpu.VMEM((TPW,), F32),
    ] + [pltpu.SemaphoreType.DMA] * 13,
)(functools.partial(_agg_body, True))


TB = 1000


def _dinv_from(dg_ref):
    deg = dg_ref[0] + dg_ref[1] + 1.0
    return lax.rsqrt(deg)


def _tc1_body(x_ref, w_ref, dg_ref, g_ref, dinv_ref):
    dinv = _dinv_from(dg_ref)
    h = jnp.dot(x_ref[...], w_ref[...], preferred_element_type=F32)
    g_ref[...] = h * dinv
    dinv_ref[...] = dinv


def _tc2_body(g_in_ref, p_ref, dg_ref, b_ref, w_ref, out_ref):
    dinv = _dinv_from(dg_ref)
    su = p_ref[0] + p_ref[1] + g_in_ref[...]
    h = jnp.maximum(dinv * su + b_ref[...], 0.0)
    out_ref[...] = dinv * jnp.dot(h, w_ref[...], preferred_element_type=F32)


def _tc3_body(g_in_ref, p_ref, dg_ref, wp_ref, b2_ref, w3_ref, b3_ref,
              out_ref, acc):
    i = pl.program_id(0)

    @pl.when(i == 0)
    def _():
        acc[...] = jnp.zeros((8, D), F32)

    dinv = _dinv_from(dg_ref)
    su = p_ref[0] + p_ref[1] + g_in_ref[...]
    h3 = jnp.maximum(dinv * su + b2_ref[...], 0.0)
    v = dinv * (wp_ref[0] + wp_ref[1] + dinv)
    acc[0:1, :] = acc[0:1, :] + jnp.sum(v * h3, axis=0, keepdims=True)

    @pl.when(i == pl.num_programs(0) - 1)
    def _():
        out_ref[...] = (
            jnp.dot(acc[0:1, :] / float(N), w3_ref[...],
                    preferred_element_type=F32) + b3_ref[...]
        )


def _row_spec(block):
    return pl.BlockSpec(block, lambda i: (i, 0))


def _const_spec(block):
    return pl.BlockSpec(block, lambda i: (0, 0))


_p_spec = pl.BlockSpec((2, TB, D), lambda i: (0, i, 0))
_v_spec = pl.BlockSpec((2, TB, 1), lambda i: (0, i, 0))

_tc1 = pl.pallas_call(
    _tc1_body,
    grid=(N // TB,),
    in_specs=[_row_spec((TB, D)), _const_spec((D, D)), _v_spec],
    out_specs=[_row_spec((TB, D)), _row_spec((TB, 1))],
    out_shape=[jax.ShapeDtypeStruct((N, D), F32),
               jax.ShapeDtypeStruct((N, 1), F32)],
)

_tc2 = pl.pallas_call(
    _tc2_body,
    grid=(N // TB,),
    in_specs=[_row_spec((TB, D)), _p_spec, _v_spec,
              _const_spec((1, D)), _const_spec((D, D))],
    out_specs=_row_spec((TB, D)),
    out_shape=jax.ShapeDtypeStruct((N, D), F32),
)

_tc3 = pl.pallas_call(
    _tc3_body,
    grid=(N // TB,),
    in_specs=[_row_spec((TB, D)), _p_spec, _v_spec, _v_spec,
              _const_spec((1, D)), _const_spec((D, D)), _const_spec((1, D))],
    out_specs=_const_spec((1, D)),
    out_shape=jax.ShapeDtypeStruct((1, D), F32),
    scratch_shapes=[pltpu.VMEM((8, D), F32)],
)


def kernel(x, edge_index, W1, b1, W2, b2, W3, b3):
    roww = edge_index[0]
    colw = edge_index[1]

    degp = _deg(colw).reshape(NC, NPAD, 1)

    g1, dinv = _tc1(x, W1, degp)

    p1 = _agg(g1, roww, colw)
    g2 = _tc2(g1, p1, degp, b1.reshape(1, D), W2)
    p2, wp = _aggw(g2, roww, colw, dinv.reshape(N))

    return _tc3(g2, p2, degp, wp.reshape(NC, NPAD, 1),
                b2.reshape(1, D), W3, b3.reshape(1, D))


# speedup vs baseline: 1.0917x; 1.0519x over previous
import functools

import jax
import jax.numpy as jnp
from jax import lax
from jax.experimental import pallas as pl
from jax.experimental.pallas import tpu as pltpu
from jax.experimental.pallas import tpu_sc as plsc

F32 = jnp.float32

N = 10000
E = 320000
D = 128

NC = 2
NS = 16
NW = NC * NS
EPW = E // NW
K = 112
CWF = EPW // K
KT = EPW - CWF * K
CW = CWF + 1
UNROLL = 6
CWU = 84
NPAD = 10240
NPAD2 = 10112
TPW = NPAD // NS
TPW2 = NPAD2 // NS

_mesh = plsc.VectorSubcoreMesh(core_axis_name="c", subcore_axis_name="s")


def _zero_rows(buf):
    z = jnp.zeros((16,), F32)

    def body(i, _):
        for k in range(buf.shape[1] // 16):
            buf[i, pl.ds(k * 16, 16)] = z
        return 0

    lax.fori_loop(0, buf.shape[0], body, 0)


def _zero_1d(buf):
    z = jnp.zeros((16,), F32)

    def body(i, _):
        buf[pl.ds(i * 16, 16)] = z
        return 0

    lax.fori_loop(0, buf.shape[0] // 16, body, 0)


@functools.partial(
    pl.kernel,
    out_type=jax.ShapeDtypeStruct((NC, NPAD), F32),
    mesh=_mesh,
    scratch_types=[
        pltpu.VMEM((CWF, K), jnp.int32),
        pltpu.VMEM((1, KT), jnp.int32),
        pltpu.VMEM((K,), F32),
        pltpu.VMEM((TPW,), F32),
        pltpu.VMEM_SHARED((NPAD,), F32),
        pltpu.SemaphoreType.DMA,
    ],
)
def _deg(col_hbm, out_hbm, colv, ctail, ones, zbuf, dacc, isem):
    c = lax.axis_index("c")
    s = lax.axis_index("s")
    wid = s * NC + c

    def idx_cp(j):
        off = pl.multiple_of(wid * EPW + j * K, 8)
        return pltpu.make_async_copy(col_hbm.at[pl.ds(off, K)], colv.at[j],
                                     isem)

    for j in range(CWF):
        idx_cp(j).start()
    toff = pl.multiple_of(wid * EPW + CWF * K, 8)
    pltpu.async_copy(col_hbm.at[pl.ds(toff, KT)], ctail.at[0], isem)
    for k in range(K // 16):
        ones[pl.ds(k * 16, 16)] = jnp.ones((16,), F32)
    _zero_1d(zbuf)
    pltpu.sync_copy(zbuf, dacc.at[pl.ds(s * TPW, TPW)])
    for j in range(CWF):
        idx_cp(j).wait()
    pltpu.make_async_copy(col_hbm.at[pl.ds(toff, KT)], ctail.at[0], isem).wait()
    plsc.subcore_barrier()

    def body(j, _):
        pltpu.sync_copy(ones, dacc.at[colv.at[j]], add=True)
        return 0

    lax.fori_loop(0, CWF, body, 0)
    pltpu.sync_copy(ones.at[pl.ds(0, KT)], dacc.at[ctail.at[0]], add=True)
    plsc.subcore_barrier()
    pltpu.sync_copy(dacc.at[pl.ds(s * TPW, TPW)], out_hbm.at[c, pl.ds(s * TPW, TPW)])


def _agg_body(with_wsum, g_hbm, row_hbm, col_hbm, *args):
    if with_wsum:
        (dinv_hbm, out_hbm, wout_hbm, ribuf, cibuf, rtail, ctail, b0, b1, b2,
         acc, dsp, wacc, vals, zbuf, s0, s1, s2, i0, i1, i2, i3, i4, i5,
         tsem, c0, c1, c2) = args
    else:
        (out_hbm, ribuf, cibuf, rtail, ctail, b0, b1, b2, acc, s0, s1, s2,
         i0, i1, i2, i3, i4, i5, tsem, c0, c1, c2) = args
    bufs = (b0, b1, b2)
    gsem = (s0, s1, s2)
    isem = (i0, i1, i2, i3, i4, i5)
    ssem = (c0, c1, c2)
    c = lax.axis_index("c")
    s = lax.axis_index("s")
    wid = s * NC + c

    _zero_rows(b0)
    nfull = TPW2 // K
    for b in range(nfull):
        pltpu.sync_copy(b0, acc.at[pl.ds(s * TPW2 + b * K, K)])
    rem = TPW2 - nfull * K
    if rem:
        pltpu.sync_copy(b0.at[pl.ds(0, rem)],
                        acc.at[pl.ds(s * TPW2 + nfull * K, rem)])
    if with_wsum:
        _zero_1d(zbuf)
        pltpu.sync_copy(zbuf, wacc.at[pl.ds(s * TPW, TPW)])

        @pl.when(s == 0)
        def _():
            pltpu.sync_copy(dinv_hbm, dsp)

    plsc.subcore_barrier()

    def _ioff(j):
        return pl.multiple_of(wid * EPW + j * K, 8)

    def idx_start(j, m):
        off = _ioff(j)
        pltpu.make_async_copy(row_hbm.at[pl.ds(off, K)], ribuf.at[m], isem[m]).start()
        pltpu.make_async_copy(col_hbm.at[pl.ds(off, K)], cibuf.at[m], isem[m]).start()

    def idx_wait(j, m):
        off = _ioff(j)
        pltpu.make_async_copy(row_hbm.at[pl.ds(off, K)], ribuf.at[m], isem[m]).wait()
        pltpu.make_async_copy(col_hbm.at[pl.ds(off, K)], cibuf.at[m], isem[m]).wait()

    def g_cp(t, m):
        return pltpu.make_async_copy(g_hbm.at[ribuf.at[m]], bufs[t], gsem[t])

    def sc_start(t, m):
        pltpu.async_copy(bufs[t], acc.at[cibuf.at[m]], ssem[t], add=True)

    def sc_wait(t, m):
        pltpu.make_async_copy(bufs[t], acc.at[cibuf.at[m]], ssem[t]).wait()

    for j in range(4):
        idx_start(j, j)
    idx_wait(0, 0)
    g_cp(0, 0).start()
    idx_wait(1, 1)
    g_cp(1, 1).start()

    toff = pl.multiple_of(wid * EPW + CWF * K, 8)

    def tidx_start():
        pltpu.make_async_copy(row_hbm.at[pl.ds(toff, KT)], rtail.at[0], tsem).start()
        pltpu.make_async_copy(col_hbm.at[pl.ds(toff, KT)], ctail.at[0], tsem).start()

    def tidx_wait():
        pltpu.make_async_copy(row_hbm.at[pl.ds(toff, KT)], rtail.at[0], tsem).wait()
        pltpu.make_async_copy(col_hbm.at[pl.ds(toff, KT)], ctail.at[0], tsem).wait()

    def tg_cp(t):
        return pltpu.make_async_copy(g_hbm.at[rtail.at[0]],
                                     bufs[t].at[pl.ds(0, KT)], gsem[t])

    def tsc_start(t):
        pltpu.async_copy(bufs[t].at[pl.ds(0, KT)], acc.at[ctail.at[0]],
                         ssem[t], add=True)

    def tsc_wait(t):
        pltpu.make_async_copy(bufs[t].at[pl.ds(0, KT)], acc.at[ctail.at[0]],
                              ssem[t]).wait()

    def step(j, u, cond, gnext_tail=False, inext_tail=False):
        t = u % 3
        tn = (u + 2) % 3
        mn = (u + 2) % 6

        def ab():
            def sw():
                sc_wait(tn, (u + 5) % 6)

            if gnext_tail:
                tidx_wait()
                cond(j >= 1, sw)
                tg_cp(tn).start()
            else:
                idx_wait(j + 2, mn)
                cond(j >= 1, sw)
                g_cp(tn, mn).start()

        cond(j + 2 < CW, ab)
        g_cp(t, u).wait()
        sc_start(t, u)
        if with_wsum:
            pltpu.sync_copy(dsp.at[cibuf.at[u]], vals)
            pltpu.sync_copy(vals, wacc.at[ribuf.at[u]], add=True)
        if inext_tail:
            tidx_start()
        else:
            cond(j + 4 < CW - 1, lambda: idx_start(j + 4, (u + 4) % 6))

    def dyn_cond(pred, fn):
        pl.when(pred)(fn)

    def static_cond(pred, fn):
        if pred:
            fn()

    def body(k, _):
        j0 = UNROLL * k
        for u in range(UNROLL):
            step(j0 + u, u, dyn_cond)
        return 0

    lax.fori_loop(0, CWU // UNROLL, body, 0)
    for j in range(CWU, CWF):
        step(j, j % UNROLL, static_cond,
             gnext_tail=(j + 2 == CW - 1), inext_tail=(j + 4 == CW - 1))
    tt = (CW - 1) % 3
    tg_cp(tt).wait()
    tsc_start(tt)
    if with_wsum:
        pltpu.sync_copy(dsp.at[ctail.at[0]], vals.at[pl.ds(0, KT)])
        pltpu.sync_copy(vals.at[pl.ds(0, KT)], wacc.at[rtail.at[0]], add=True)
    for j in range(CW - 3, CW - 1):
        sc_wait(j % 3, j % 6)
    tsc_wait(tt)

    plsc.subcore_barrier()
    pltpu.sync_copy(acc.at[pl.ds(s * TPW2, TPW2)],
                    out_hbm.at[c, pl.ds(s * TPW2, TPW2)])
    if with_wsum:
        pltpu.sync_copy(wacc.at[pl.ds(s * TPW, TPW)],
                        wout_hbm.at[c, pl.ds(s * TPW, TPW)])


_agg = functools.partial(
    pl.kernel,
    out_type=jax.ShapeDtypeStruct((NC, NPAD2, D), F32),
    mesh=_mesh,
    scratch_types=[
        pltpu.VMEM((6, K), jnp.int32),
        pltpu.VMEM((6, K), jnp.int32),
        pltpu.VMEM((1, KT), jnp.int32),
        pltpu.VMEM((1, KT), jnp.int32),
        pltpu.VMEM((K, D), F32),
        pltpu.VMEM((K, D), F32),
        pltpu.VMEM((K, D), F32),
        pltpu.VMEM_SHARED((NPAD2, D), F32),
    ] + [pltpu.SemaphoreType.DMA] * 13,
)(functools.partial(_agg_body, False))

_aggw = functools.partial(
    pl.kernel,
    out_type=[jax.ShapeDtypeStruct((NC, NPAD2, D), F32),
              jax.ShapeDtypeStruct((NC, NPAD), F32)],
    mesh=_mesh,
    scratch_types=[
        pltpu.VMEM((6, K), jnp.int32),
        pltpu.VMEM((6, K), jnp.int32),
        pltpu.VMEM((1, KT), jnp.int32),
        pltpu.VMEM((1, KT), jnp.int32),
        pltpu.VMEM((K, D), F32),
        pltpu.VMEM((K, D), F32),
        pltpu.VMEM((K, D), F32),
        pltpu.VMEM_SHARED((NPAD2, D), F32),
        pltpu.VMEM_SHARED((N,), F32),
        pltpu.VMEM_SHARED((NPAD,), F32),
        pltpu.VMEM((K,), F32),
        pltpu.VMEM((TPW,), F32),
    ] + [pltpu.SemaphoreType.DMA] * 13,
)(functools.partial(_agg_body, True))


TB = 1000


def _dinv_from(dg_ref):
    deg = dg_ref[0] + dg_ref[1] + 1.0
    return lax.rsqrt(deg)


def _tc1_body(x_ref, w_ref, dg_ref, g_ref, dinv_ref):
    dinv = _dinv_from(dg_ref)
    h = jnp.dot(x_ref[...], w_ref[...], preferred_element_type=F32)
    g_ref[...] = h * dinv
    dinv_ref[...] = dinv


def _tc2_body(g_in_ref, p_ref, dg_ref, b_ref, w_ref, out_ref):
    dinv = _dinv_from(dg_ref)
    su = p_ref[0] + p_ref[1] + g_in_ref[...]
    h = jnp.maximum(dinv * su + b_ref[...], 0.0)
    out_ref[...] = dinv * jnp.dot(h, w_ref[...], preferred_element_type=F32)


def _tc3_body(g_in_ref, p_ref, dg_ref, wp_ref, b2_ref, w3_ref, b3_ref,
              out_ref, acc):
    i = pl.program_id(0)

    @pl.when(i == 0)
    def _():
        acc[...] = jnp.zeros((8, D), F32)

    dinv = _dinv_from(dg_ref)
    su = p_ref[0] + p_ref[1] + g_in_ref[...]
    h3 = jnp.maximum(dinv * su + b2_ref[...], 0.0)
    v = dinv * (wp_ref[0] + wp_ref[1] + dinv)
    acc[0:1, :] = acc[0:1, :] + jnp.sum(v * h3, axis=0, keepdims=True)

    @pl.when(i == pl.num_programs(0) - 1)
    def _():
        out_ref[...] = (
            jnp.dot(acc[0:1, :] / float(N), w3_ref[...],
                    preferred_element_type=F32) + b3_ref[...]
        )


def _row_spec(block):
    return pl.BlockSpec(block, lambda i: (i, 0))


def _const_spec(block):
    return pl.BlockSpec(block, lambda i: (0, 0))


_p_spec = pl.BlockSpec((2, TB, D), lambda i: (0, i, 0))
_v_spec = pl.BlockSpec((2, TB, 1), lambda i: (0, i, 0))

_tc1 = pl.pallas_call(
    _tc1_body,
    grid=(N // TB,),
    in_specs=[_row_spec((TB, D)), _const_spec((D, D)), _v_spec],
    out_specs=[_row_spec((TB, D)), _row_spec((TB, 1))],
    out_shape=[jax.ShapeDtypeStruct((N, D), F32),
               jax.ShapeDtypeStruct((N, 1), F32)],
)

_tc2 = pl.pallas_call(
    _tc2_body,
    grid=(N // TB,),
    in_specs=[_row_spec((TB, D)), _p_spec, _v_spec,
              _const_spec((1, D)), _const_spec((D, D))],
    out_specs=_row_spec((TB, D)),
    out_shape=jax.ShapeDtypeStruct((N, D), F32),
)

_tc3 = pl.pallas_call(
    _tc3_body,
    grid=(N // TB,),
    in_specs=[_row_spec((TB, D)), _p_spec, _v_spec, _v_spec,
              _const_spec((1, D)), _const_spec((D, D)), _const_spec((1, D))],
    out_specs=_const_spec((1, D)),
    out_shape=jax.ShapeDtypeStruct((1, D), F32),
    scratch_shapes=[pltpu.VMEM((8, D), F32)],
)


def kernel(x, edge_index, W1, b1, W2, b2, W3, b3):
    roww = edge_index[0]
    colw = edge_index[1]

    degp = _deg(colw).reshape(NC, NPAD, 1)

    g1, dinv = _tc1(x, W1, degp)

    p1 = _agg(g1, roww, colw)
    g2 = _tc2(g1, p1, degp, b1.reshape(1, D), W2)
    p2, wp = _aggw(g2, roww, colw, dinv.reshape(N))

    return _tc3(g2, p2, degp, wp.reshape(NC, NPAD, 1),
                b2.reshape(1, D), W3, b3.reshape(1, D))
